# baseline (speedup 1.0000x reference)
import jax
import jax.numpy as jnp
from jax.experimental import pallas as pl
from jax.experimental.pallas import tpu as pltpu

T, B, C, H, W, E, TOPK = 4, 16, 128, 16, 16, 8, 2
HW = H * W
NR = B * HW
N = T * NR
EPS = 1e-5


def _lif_spikes(xs, inv_tau):
    f32 = jnp.float32
    one = jnp.float32(1.0)
    zero = jnp.float32(0.0)
    v = jnp.zeros((NR, C), f32)
    sps = []
    for t in range(T):
        v = v + (xs[t] - v) * inv_tau
        mask = v >= 1.0
        sps.append(jnp.where(mask, one, zero))
        v = jnp.where(mask, zero, v)
    return sps


def _bn_coeffs(sps, w_raw, g, bt):
    f32 = jnp.float32
    bf16 = jnp.bfloat16
    ones_row = jnp.ones((8, NR), bf16)
    mom = jnp.zeros((C, C), f32)
    csum = jnp.zeros((8, C), f32)
    for t in range(T):
        spb = sps[t].astype(bf16)
        mom = mom + jax.lax.dot_general(
            spb, spb, (((0,), (0,)), ((), ())), preferred_element_type=f32)
        csum = csum + jnp.dot(ones_row, spb, preferred_element_type=f32)
    csum1 = csum[0:1]
    mean = jnp.dot(csum1, w_raw, preferred_element_type=f32) / N
    a = jnp.dot(mom, w_raw, preferred_element_type=f32)
    q = jnp.sum(w_raw * a, axis=0, keepdims=True) / N
    sc = g * jax.lax.rsqrt(q - mean * mean + EPS)
    sh = bt - mean * sc
    return sc, sh


def _fused(x_ref, w1_ref, b1_ref, g1_ref, bt1_ref, w2_ref, b2_ref, g2_ref,
           bt2_ref, wr_ref, brv_ref, gr_ref, btr_ref, taus_ref, o_ref):
    f32 = jnp.float32
    one = jnp.float32(1.0)
    zero = jnp.float32(0.0)
    xs = [x_ref[t] for t in range(T)]

    wr = wr_ref[...]
    v = jnp.zeros((NR, C), f32)
    ssum = jnp.zeros((1, E), f32)
    ssq = jnp.zeros((1, E), f32)
    macc = jnp.zeros((B, E), f32)
    for t in range(T):
        v = v + (xs[t] - v) * 0.5
        mask = v >= 1.0
        sp = jnp.where(mask, one, zero)
        v = jnp.where(mask, zero, v)
        r = jnp.dot(sp, wr, preferred_element_type=f32)
        ssum = ssum + jnp.sum(r, axis=0, keepdims=True)
        ssq = ssq + jnp.sum(r * r, axis=0, keepdims=True)
        macc = macc + jnp.sum(r.reshape(B, HW, E), axis=1)
    mu = ssum / N
    var = ssq / N - mu * mu
    logits = (macc / (T * HW) - mu) * jax.lax.rsqrt(var + EPS) * gr_ref[...] \
        + btr_ref[...]

    lmax = jnp.max(logits, axis=1, keepdims=True)
    ex = jnp.exp(logits - lmax)
    p = ex / jnp.sum(ex, axis=1, keepdims=True)
    ii = jax.lax.broadcasted_iota(jnp.int32, (B, E), 1)
    p1 = jnp.max(p, axis=1, keepdims=True)
    i1 = jnp.min(jnp.where(p == p1, ii, E), axis=1, keepdims=True)
    pm = jnp.where(ii == i1, -jnp.inf, p)
    p2 = jnp.max(pm, axis=1, keepdims=True)
    i2 = jnp.min(jnp.where(pm == p2, ii, E), axis=1, keepdims=True)
    keep = (ii == i1) | (ii == i2)
    wdense = jnp.where(keep, p, 0.0) / (p1 + p2)
    wsum = jnp.sum(wdense, axis=1, keepdims=True)

    acc = [jnp.zeros((NR, C), f32) for _ in range(T)]
    shift = jnp.zeros((B, C), f32)
    for e in range(E):
        inv_tau = 1.0 / taus_ref[0, e]
        w1e = w1_ref[e]
        w2e = w2_ref[e]
        g1e = g1_ref[e:e + 1]
        bt1e = bt1_ref[e:e + 1]
        g2e = g2_ref[e:e + 1]
        bt2e = bt2_ref[e:e + 1]
        wcol = wdense[:, e:e + 1]
        we3 = wcol.reshape(B, 1, 1)

        sp1 = _lif_spikes(xs, inv_tau)
        sc1, sh1 = _bn_coeffs(sp1, w1e, g1e, bt1e)
        w1f = w1e * sc1
        d1 = [jnp.dot(sp1[t], w1f, preferred_element_type=f32)
              for t in range(T)]
        hA = [xs[t] + d1[t] + sh1 for t in range(T)]

        sp2 = _lif_spikes(hA, inv_tau)
        sc2, sh2 = _bn_coeffs(sp2, w2e, g2e, bt2e)
        w2f = w2e * sc2
        shift = shift + wcol * (sh1 + sh2)
        for t in range(T):
            d2 = jnp.dot(sp2[t], w2f, preferred_element_type=f32)
            a3 = acc[t].reshape(B, HW, C) \
                + (d1[t] + d2).reshape(B, HW, C) * we3
            acc[t] = a3.reshape(NR, C)

    swb = wsum.reshape(B, 1, 1)
    shb = shift.reshape(B, 1, C)
    for t in range(T):
        o3 = xs[t].reshape(B, HW, C) * swb + acc[t].reshape(B, HW, C) + shb
        o_ref[t] = o3.reshape(NR, C)


def kernel(x, W1, b1, g1, bt1, W2, b2, g2, bt2, Wr, brv, gr, btr, taus):
    xt = x.transpose(0, 1, 3, 4, 2).reshape(T, NR, C)
    out = pl.pallas_call(
        _fused,
        out_shape=jax.ShapeDtypeStruct((T, NR, C), x.dtype),
        compiler_params=pltpu.CompilerParams(
            vmem_limit_bytes=128 * 1024 * 1024),
    )(xt,
      W1.transpose(0, 2, 1), b1, g1, bt1,
      W2.transpose(0, 2, 1), b2, g2, bt2,
      Wr.T, brv.reshape(1, E), gr.reshape(1, E), btr.reshape(1, E),
      taus.reshape(1, E))
    return out.reshape(T, B, H, W, C).transpose(0, 1, 4, 2, 3)


# speedup vs baseline: 1.1621x; 1.1621x over previous
import jax
import jax.numpy as jnp
from jax.experimental import pallas as pl
from jax.experimental.pallas import tpu as pltpu

T, B, C, H, W, E, TOPK = 4, 16, 128, 16, 16, 8, 2
HW = H * W
NR = B * HW
N = T * NR
EPS = 1e-5


def _fused(x_ref, w1_ref, b1_ref, g1_ref, bt1_ref, w2_ref, b2_ref, g2_ref,
           bt2_ref, wr_ref, brv_ref, gr_ref, btr_ref, taus_ref, o_ref):
    f32 = jnp.float32
    one = jnp.float32(1.0)
    zero = jnp.float32(0.0)
    xs = [x_ref[t] for t in range(T)]

    wr = wr_ref[...]
    v = jnp.zeros((NR, C), f32)
    ssum = jnp.zeros((1, E), f32)
    ssq = jnp.zeros((1, E), f32)
    macc = jnp.zeros((B, E), f32)
    for t in range(T):
        v = v + (xs[t] - v) * 0.5
        mask = v >= 1.0
        sp = jnp.where(mask, one, zero)
        v = jnp.where(mask, zero, v)
        r = jnp.dot(sp, wr, preferred_element_type=f32)
        ssum = ssum + jnp.sum(r, axis=0, keepdims=True)
        ssq = ssq + jnp.sum(r * r, axis=0, keepdims=True)
        macc = macc + jnp.sum(r.reshape(B, HW, E), axis=1)
    mu = ssum / N
    var = ssq / N - mu * mu
    logits = (macc / (T * HW) - mu) * jax.lax.rsqrt(var + EPS) * gr_ref[...] \
        + btr_ref[...]

    lmax = jnp.max(logits, axis=1, keepdims=True)
    ex = jnp.exp(logits - lmax)
    p = ex / jnp.sum(ex, axis=1, keepdims=True)
    ii = jax.lax.broadcasted_iota(jnp.int32, (B, E), 1)
    p1 = jnp.max(p, axis=1, keepdims=True)
    i1 = jnp.min(jnp.where(p == p1, ii, E), axis=1, keepdims=True)
    pm = jnp.where(ii == i1, -jnp.inf, p)
    p2 = jnp.max(pm, axis=1, keepdims=True)
    i2 = jnp.min(jnp.where(pm == p2, ii, E), axis=1, keepdims=True)
    keep = (ii == i1) | (ii == i2)
    wdense = jnp.where(keep, p, 0.0) / (p1 + p2)
    wsum = jnp.sum(wdense, axis=1, keepdims=True)

    acc = [jnp.zeros((NR, C), f32) for _ in range(T)]
    shift = jnp.zeros((B, C), f32)
    for e in range(E):
        inv_tau = 1.0 / taus_ref[0, e]
        w1e = w1_ref[e]
        w2e = w2_ref[e]
        g1e = g1_ref[e:e + 1]
        bt1e = bt1_ref[e:e + 1]
        g2e = g2_ref[e:e + 1]
        bt2e = bt2_ref[e:e + 1]

        v = jnp.zeros((NR, C), f32)
        h1 = []
        s1 = jnp.zeros((1, C), f32)
        q1 = jnp.zeros((1, C), f32)
        for t in range(T):
            v = v + (xs[t] - v) * inv_tau
            mask = v >= 1.0
            sp = jnp.where(mask, one, zero)
            v = jnp.where(mask, zero, v)
            h = jnp.dot(sp, w1e, preferred_element_type=f32)
            s1 = s1 + jnp.sum(h, axis=0, keepdims=True)
            q1 = q1 + jnp.sum(h * h, axis=0, keepdims=True)
            h1.append(h)
        mean1 = s1 / N
        sc1 = g1e * jax.lax.rsqrt(q1 / N - mean1 * mean1 + EPS)
        sh1 = bt1e - mean1 * sc1
        hA = [xs[t] + h1[t] * sc1 + sh1 for t in range(T)]

        v = jnp.zeros((NR, C), f32)
        h2 = []
        s2 = jnp.zeros((1, C), f32)
        q2 = jnp.zeros((1, C), f32)
        for t in range(T):
            v = v + (hA[t] - v) * inv_tau
            mask = v >= 1.0
            sp = jnp.where(mask, one, zero)
            v = jnp.where(mask, zero, v)
            h = jnp.dot(sp, w2e, preferred_element_type=f32)
            s2 = s2 + jnp.sum(h, axis=0, keepdims=True)
            q2 = q2 + jnp.sum(h * h, axis=0, keepdims=True)
            h2.append(h)
        mean2 = s2 / N
        sc2 = g2e * jax.lax.rsqrt(q2 / N - mean2 * mean2 + EPS)
        sh2 = bt2e - mean2 * sc2

        wcol = wdense[:, e:e + 1]
        shift = shift + wcol * (sh1 + sh2)
        ws1 = (wcol.reshape(B, 1, 1) * sc1.reshape(1, 1, C))
        ws2 = (wcol.reshape(B, 1, 1) * sc2.reshape(1, 1, C))
        for t in range(T):
            a3 = acc[t].reshape(B, HW, C)
            a3 = a3 + h1[t].reshape(B, HW, C) * ws1 \
                    + h2[t].reshape(B, HW, C) * ws2
            acc[t] = a3.reshape(NR, C)

    swb = wsum.reshape(B, 1, 1)
    shb = shift.reshape(B, 1, C)
    for t in range(T):
        o3 = xs[t].reshape(B, HW, C) * swb + acc[t].reshape(B, HW, C) + shb
        o_ref[t] = o3.reshape(NR, C)


def kernel(x, W1, b1, g1, bt1, W2, b2, g2, bt2, Wr, brv, gr, btr, taus):
    xt = x.transpose(0, 1, 3, 4, 2).reshape(T, NR, C)
    out = pl.pallas_call(
        _fused,
        out_shape=jax.ShapeDtypeStruct((T, NR, C), x.dtype),
        compiler_params=pltpu.CompilerParams(
            vmem_limit_bytes=128 * 1024 * 1024),
    )(xt,
      W1.transpose(0, 2, 1), b1, g1, bt1,
      W2.transpose(0, 2, 1), b2, g2, bt2,
      Wr.T, brv.reshape(1, E), gr.reshape(1, E), btr.reshape(1, E),
      taus.reshape(1, E))
    return out.reshape(T, B, H, W, C).transpose(0, 1, 4, 2, 3)
